# baseline (speedup 1.0000x reference)
import functools

import jax
import jax.numpy as jnp
from jax.experimental import pallas as pl
from jax.experimental.pallas import tpu as pltpu

DIM = 32
K = 512


def _vq_block(x_ref, embed_ref, q_ref, ind_ref):
    xb = x_ref[...]
    G, T, _ = xb.shape
    N = G * T
    xf = xb.reshape(N, DIM)
    e = embed_ref[...]
    xe = jax.lax.dot_general(
        xf, e, (((1,), (1,)), ((), ())),
        preferred_element_type=jnp.float32,
    )
    x2 = jnp.sum(xf * xf, axis=1, keepdims=True)
    e2 = jnp.sum(e * e, axis=1)[None, :]
    b = x2 - 2.0 * xe + e2
    m_row = jnp.min(b, axis=-1, keepdims=True)
    onehot = (b == m_row).astype(jnp.bfloat16)
    e_hi = e.astype(jnp.bfloat16)
    e_lo = (e - e_hi.astype(jnp.float32)).astype(jnp.bfloat16)
    k_iota = jax.lax.broadcasted_iota(jnp.int32, (K, 1), 0)
    i_hi = (k_iota // 16).astype(jnp.bfloat16)
    i_lo = (k_iota % 16).astype(jnp.bfloat16)
    e_cat = jnp.concatenate([e_hi, e_lo, i_hi, i_lo], axis=1)
    q2 = jax.lax.dot_general(
        onehot, e_cat, (((1,), (0,)), ((), ())),
        preferred_element_type=jnp.float32,
    )
    q = q2[:, :DIM] + q2[:, DIM:2 * DIM]
    ind = (
        16.0 * q2[:, 2 * DIM] + q2[:, 2 * DIM + 1]
    ).astype(jnp.int32)
    ind_ref[...] = ind.reshape(G, 1, T)
    q_ref[...] = q.reshape(G, T, DIM)


@functools.partial(jax.jit, static_argnames=("g",))
def _vq(x, embed, g=8):
    B, T, D = x.shape
    grid = (B // g,)
    q, ind = pl.pallas_call(
        _vq_block,
        grid=grid,
        in_specs=[
            pl.BlockSpec((g, T, D), lambda i: (i, 0, 0)),
            pl.BlockSpec((K, D), lambda i: (0, 0)),
        ],
        out_specs=[
            pl.BlockSpec((g, T, D), lambda i: (i, 0, 0)),
            pl.BlockSpec((g, 1, T), lambda i: (i, 0, 0)),
        ],
        out_shape=[
            jax.ShapeDtypeStruct((B, T, D), jnp.float32),
            jax.ShapeDtypeStruct((B, 1, T), jnp.int32),
        ],
    )(x, embed)
    return q, ind.reshape(B, T)


def kernel(x, embed):
    quantize, embed_ind = _vq(x, embed)
    return (quantize, embed_ind)
